# baseline (speedup 1.0000x reference)
import functools

import jax
import jax.numpy as jnp
from jax.experimental import pallas as pl
from jax.experimental.pallas import tpu as pltpu

N_SITES = 784
D = 128
B = 256
S = 56
NBLK = N_SITES // S


def _row_sort_perm():
    row = jax.lax.broadcasted_iota(jnp.int32, (2 * D, 2 * D), 0)
    col = jax.lax.broadcasted_iota(jnp.int32, (2 * D, 2 * D), 1)
    return jnp.where(2 * (row % D) + row // D == col, 1.0, 0.0)


def _mps_body(m2_ref, xft_ref, out_ref, alt_ref, gl_ref):
    j = pl.program_id(0)

    @pl.when(j == 0)
    def _init():
        row = jax.lax.broadcasted_iota(jnp.int32, (D, B), 0)
        alt_ref[...] = jnp.where(row == 0, 1.0, 0.0)
        rowg = jax.lax.broadcasted_iota(jnp.int32, (D, D), 0)
        colg = jax.lax.broadcasted_iota(jnp.int32, (D, D), 1)
        gl_ref[...] = jnp.where((rowg == 0) & (colg == 0), 1.0, 0.0)

    perm = _row_sort_perm().astype(jnp.bfloat16)

    xblk = xft_ref[...]
    e0b = jnp.cos(0.5 * jnp.pi * xblk)
    e1b = jnp.sin(0.5 * jnp.pi * xblk)

    bf = jnp.bfloat16
    alt = alt_ref[...]
    gl = gl_ref[...]
    for t in range(S // 2):
        s0, s1 = 2 * t, 2 * t + 1
        slab0 = m2_ref[2 * D * s0:2 * D * (s0 + 1), :].astype(bf)
        slab1 = m2_ref[2 * D * s1:2 * D * (s1 + 1), :].astype(bf)
        mv0 = jnp.dot(perm, slab0,
                      preferred_element_type=jnp.float32).astype(bf)
        mv1 = jnp.dot(perm, slab1,
                      preferred_element_type=jnp.float32).astype(bf)
        mcat1 = jnp.concatenate([mv1[:D], mv1[D:]], axis=1)
        pbig = jnp.dot(mv0, mcat1,
                       preferred_element_type=jnp.float32).astype(bf)
        p00, p01 = pbig[:D, :D], pbig[:D, D:]
        p10, p11 = pbig[D:, :D], pbig[D:, D:]
        pv4 = jnp.concatenate([p00, p01, p10, p11], axis=0)
        pcat4 = jnp.concatenate([p00, p01, p10, p11], axis=1)

        ep = (e0b[s0:s0 + 1], e1b[s0:s0 + 1])
        eq = (e0b[s1:s1 + 1], e1b[s1:s1 + 1])
        vb4 = jnp.concatenate(
            [alt * (ep[0] * eq[0]), alt * (ep[0] * eq[1]),
             alt * (ep[1] * eq[0]), alt * (ep[1] * eq[1])],
            axis=0).astype(bf)
        alt = jax.lax.dot_general(
            pv4, vb4, (((0,), (0,)), ((), ())),
            preferred_element_type=jnp.float32)

        w = jnp.dot(gl.astype(bf), pcat4,
                    preferred_element_type=jnp.float32)
        wv = jnp.concatenate([w[:, :D], w[:, D:2 * D],
                              w[:, 2 * D:3 * D], w[:, 3 * D:]],
                             axis=0).astype(bf)
        gl = jax.lax.dot_general(
            wv, pv4, (((0,), (0,)), ((), ())),
            preferred_element_type=jnp.float32)
    alt_ref[...] = alt
    gl_ref[...] = gl

    @pl.when(j == NBLK - 1)
    def _():
        out_ref[0] = alt
        out_ref[1, :, :D] = gl


@functools.partial(jax.jit, static_argnames=("interpret",))
def kernel(x, MPS, interpret=False):
    xft = x.reshape(B, -1).T
    m2 = MPS.transpose(0, 1, 3, 2).reshape(N_SITES * 2 * D, D)

    buf = pl.pallas_call(
        _mps_body,
        grid=(NBLK,),
        in_specs=[
            pl.BlockSpec((S * 2 * D, D), lambda j: (j, 0)),
            pl.BlockSpec((S, B), lambda j: (j, 0)),
        ],
        out_specs=pl.BlockSpec((2, D, B), lambda j: (0, 0, 0)),
        out_shape=jax.ShapeDtypeStruct((2, D, B), jnp.float32),
        scratch_shapes=[
            pltpu.VMEM((D, B), jnp.float32),
            pltpu.VMEM((D, D), jnp.float32),
        ],
        compiler_params=pltpu.CompilerParams(
            dimension_semantics=("arbitrary",),
        ),
        interpret=interpret,
    )(m2, xft)

    amp = buf[0, 0, :]
    norm_sq = buf[1, 0, 0]
    return amp * amp / norm_sq


# speedup vs baseline: 1.2516x; 1.2516x over previous
import functools

import jax
import jax.numpy as jnp
from jax.experimental import pallas as pl
from jax.experimental.pallas import tpu as pltpu

N_SITES = 784
D = 128
B = 256
S = 56
NBLK = N_SITES // S


def _row_sort_perm():
    row = jax.lax.broadcasted_iota(jnp.int32, (2 * D, 2 * D), 0)
    col = jax.lax.broadcasted_iota(jnp.int32, (2 * D, 2 * D), 1)
    return jnp.where(2 * (row % D) + row // D == col, 1.0, 0.0)


def _mps_body(m2_ref, xft_ref, out_ref, alt_ref, gl_ref):
    j = pl.program_id(0)

    @pl.when(j == 0)
    def _init():
        row = jax.lax.broadcasted_iota(jnp.int32, (D, B), 0)
        alt_ref[...] = jnp.where(row == 0, 1.0, 0.0)
        rowg = jax.lax.broadcasted_iota(jnp.int32, (D, D), 0)
        colg = jax.lax.broadcasted_iota(jnp.int32, (D, D), 1)
        gl_ref[...] = jnp.where((rowg == 0) & (colg == 0), 1.0, 0.0)

    perm = _row_sort_perm().astype(jnp.bfloat16)

    xblk = xft_ref[...]
    e0b = jnp.cos(0.5 * jnp.pi * xblk)
    e1b = jnp.sin(0.5 * jnp.pi * xblk)

    bf = jnp.bfloat16
    alt = alt_ref[...]
    gl = gl_ref[...]
    for s in range(S):
        slab = m2_ref[2 * D * s:2 * D * (s + 1), :].astype(bf)
        mv = jnp.dot(perm, slab,
                     preferred_element_type=jnp.float32).astype(bf)
        mcat = jnp.concatenate([mv[:D], mv[D:]], axis=1)

        vb = jnp.concatenate([alt * e0b[s:s + 1], alt * e1b[s:s + 1]],
                             axis=0).astype(bf)
        alt = jax.lax.dot_general(
            mv, vb, (((0,), (0,)), ((), ())),
            preferred_element_type=jnp.float32)

        w = jnp.dot(gl.astype(bf), mcat,
                    preferred_element_type=jnp.float32)
        wv = jnp.concatenate([w[:, :D], w[:, D:]], axis=0).astype(bf)
        gl = jax.lax.dot_general(
            wv, mv, (((0,), (0,)), ((), ())),
            preferred_element_type=jnp.float32)
    alt_ref[...] = alt
    gl_ref[...] = gl

    @pl.when(j == NBLK - 1)
    def _():
        out_ref[0] = alt
        out_ref[1, :, :D] = gl


@functools.partial(jax.jit, static_argnames=("interpret",))
def kernel(x, MPS, interpret=False):
    xft = x.reshape(B, -1).T
    m2 = MPS.transpose(0, 1, 3, 2).reshape(N_SITES * 2 * D, D)

    buf = pl.pallas_call(
        _mps_body,
        grid=(NBLK,),
        in_specs=[
            pl.BlockSpec((S * 2 * D, D), lambda j: (j, 0)),
            pl.BlockSpec((S, B), lambda j: (j, 0)),
        ],
        out_specs=pl.BlockSpec((2, D, B), lambda j: (0, 0, 0)),
        out_shape=jax.ShapeDtypeStruct((2, D, B), jnp.float32),
        scratch_shapes=[
            pltpu.VMEM((D, B), jnp.float32),
            pltpu.VMEM((D, D), jnp.float32),
        ],
        compiler_params=pltpu.CompilerParams(
            dimension_semantics=("arbitrary",),
        ),
        interpret=interpret,
    )(m2, xft)

    amp = buf[0, 0, :]
    norm_sq = buf[1, 0, 0]
    return amp * amp / norm_sq


# speedup vs baseline: 1.2554x; 1.0030x over previous
import functools

import jax
import jax.numpy as jnp
from jax.experimental import pallas as pl
from jax.experimental.pallas import tpu as pltpu

N_SITES = 784
D = 128
B = 256
S = 56
NBLK = N_SITES // S


def _row_sort_perm():
    row = jax.lax.broadcasted_iota(jnp.int32, (2 * D, 2 * D), 0)
    col = jax.lax.broadcasted_iota(jnp.int32, (2 * D, 2 * D), 1)
    return jnp.where(2 * (row % D) + row // D == col, 1.0, 0.0)


def _mps_body(m2_ref, xft_ref, out_ref, alt_ref, gl_ref):
    j = pl.program_id(0)

    @pl.when(j == 0)
    def _init():
        row = jax.lax.broadcasted_iota(jnp.int32, (D, B), 0)
        alt_ref[...] = jnp.where(row == 0, 1.0, 0.0)
        rowg = jax.lax.broadcasted_iota(jnp.int32, (D, D), 0)
        colg = jax.lax.broadcasted_iota(jnp.int32, (D, D), 1)
        gl_ref[...] = jnp.where((rowg == 0) & (colg == 0), 1.0, 0.0)

    perm = _row_sort_perm()

    xblk = xft_ref[...]
    e0b = jnp.cos(0.5 * jnp.pi * xblk)
    e1b = jnp.sin(0.5 * jnp.pi * xblk)

    alt = alt_ref[...]
    gl = gl_ref[...]
    for t in range(S // 2):
        s0 = 2 * t
        slab2 = jnp.concatenate(
            [m2_ref[2 * D * s0:2 * D * (s0 + 1), :],
             m2_ref[2 * D * (s0 + 1):2 * D * (s0 + 2), :]], axis=1)
        mv2 = jnp.dot(perm, slab2,
                      preferred_element_type=jnp.float32)
        for k in range(2):
            s = s0 + k
            mv = mv2[:, k * D:(k + 1) * D]
            mcat = jnp.concatenate([mv[:D], mv[D:]], axis=1)

            vb = jnp.concatenate(
                [alt * e0b[s:s + 1], alt * e1b[s:s + 1]], axis=0)
            alt = jax.lax.dot_general(
                mv, vb, (((0,), (0,)), ((), ())),
                preferred_element_type=jnp.float32)

            w = jnp.dot(gl, mcat,
                        preferred_element_type=jnp.float32)
            wv = jnp.concatenate([w[:, :D], w[:, D:]], axis=0)
            gl = jax.lax.dot_general(
                wv, mv, (((0,), (0,)), ((), ())),
                preferred_element_type=jnp.float32)
    alt_ref[...] = alt
    gl_ref[...] = gl

    @pl.when(j == NBLK - 1)
    def _():
        out_ref[0] = alt
        out_ref[1, :, :D] = gl


@functools.partial(jax.jit, static_argnames=("interpret",))
def kernel(x, MPS, interpret=False):
    xft = x.reshape(B, -1).T
    m2 = MPS.transpose(0, 1, 3, 2).reshape(N_SITES * 2 * D, D)

    buf = pl.pallas_call(
        _mps_body,
        grid=(NBLK,),
        in_specs=[
            pl.BlockSpec((S * 2 * D, D), lambda j: (j, 0)),
            pl.BlockSpec((S, B), lambda j: (j, 0)),
        ],
        out_specs=pl.BlockSpec((2, D, B), lambda j: (0, 0, 0)),
        out_shape=jax.ShapeDtypeStruct((2, D, B), jnp.float32),
        scratch_shapes=[
            pltpu.VMEM((D, B), jnp.float32),
            pltpu.VMEM((D, D), jnp.float32),
        ],
        compiler_params=pltpu.CompilerParams(
            dimension_semantics=("arbitrary",),
        ),
        interpret=interpret,
    )(m2, xft)

    amp = buf[0, 0, :]
    norm_sq = buf[1, 0, 0]
    return amp * amp / norm_sq


# speedup vs baseline: 1.2758x; 1.0163x over previous
import functools

import jax
import jax.numpy as jnp
from jax.experimental import pallas as pl
from jax.experimental.pallas import tpu as pltpu

N_SITES = 784
D = 128
B = 256
S = 112
NBLK = N_SITES // S


def _row_sort_perm():
    row = jax.lax.broadcasted_iota(jnp.int32, (2 * D, 2 * D), 0)
    col = jax.lax.broadcasted_iota(jnp.int32, (2 * D, 2 * D), 1)
    return jnp.where(2 * (row % D) + row // D == col, 1.0, 0.0)


def _mps_body(m2_ref, xft_ref, out_ref, alt_ref, gl_ref):
    j = pl.program_id(0)

    @pl.when(j == 0)
    def _init():
        row = jax.lax.broadcasted_iota(jnp.int32, (D, B), 0)
        alt_ref[...] = jnp.where(row == 0, 1.0, 0.0)
        rowg = jax.lax.broadcasted_iota(jnp.int32, (D, D), 0)
        colg = jax.lax.broadcasted_iota(jnp.int32, (D, D), 1)
        gl_ref[...] = jnp.where((rowg == 0) & (colg == 0), 1.0, 0.0)

    perm = _row_sort_perm()

    xblk = xft_ref[...]
    e0b = jnp.cos(0.5 * jnp.pi * xblk)
    e1b = jnp.sin(0.5 * jnp.pi * xblk)

    alt = alt_ref[...]
    gl = gl_ref[...]
    for s in range(S):
        slab = m2_ref[2 * D * s:2 * D * (s + 1), :]
        mv = jnp.dot(perm, slab,
                     preferred_element_type=jnp.float32)
        mcat = jnp.concatenate([mv[:D], mv[D:]], axis=1)

        vb = jnp.concatenate([alt * e0b[s:s + 1], alt * e1b[s:s + 1]], axis=0)
        alt = jax.lax.dot_general(
            mv, vb, (((0,), (0,)), ((), ())),
            preferred_element_type=jnp.float32)

        w = jnp.dot(gl, mcat,
                    preferred_element_type=jnp.float32)
        wv = jnp.concatenate([w[:, :D], w[:, D:]], axis=0)
        gl = jax.lax.dot_general(
            wv, mv, (((0,), (0,)), ((), ())),
            preferred_element_type=jnp.float32)
    alt_ref[...] = alt
    gl_ref[...] = gl

    @pl.when(j == NBLK - 1)
    def _():
        out_ref[0] = alt
        out_ref[1, :, :D] = gl


@functools.partial(jax.jit, static_argnames=("interpret",))
def kernel(x, MPS, interpret=False):
    xft = x.reshape(B, -1).T
    m2 = MPS.transpose(0, 1, 3, 2).reshape(N_SITES * 2 * D, D)

    buf = pl.pallas_call(
        _mps_body,
        grid=(NBLK,),
        in_specs=[
            pl.BlockSpec((S * 2 * D, D), lambda j: (j, 0)),
            pl.BlockSpec((S, B), lambda j: (j, 0)),
        ],
        out_specs=pl.BlockSpec((2, D, B), lambda j: (0, 0, 0)),
        out_shape=jax.ShapeDtypeStruct((2, D, B), jnp.float32),
        scratch_shapes=[
            pltpu.VMEM((D, B), jnp.float32),
            pltpu.VMEM((D, D), jnp.float32),
        ],
        compiler_params=pltpu.CompilerParams(
            dimension_semantics=("arbitrary",),
        ),
        interpret=interpret,
    )(m2, xft)

    amp = buf[0, 0, :]
    norm_sq = buf[1, 0, 0]
    return amp * amp / norm_sq


# speedup vs baseline: 1.2787x; 1.0023x over previous
import functools

import jax
import jax.numpy as jnp
from jax.experimental import pallas as pl
from jax.experimental.pallas import tpu as pltpu

N_SITES = 784
D = 128
B = 256
S = 56
NBLK = N_SITES // S


def _row_sort_perm():
    row = jax.lax.broadcasted_iota(jnp.int32, (2 * D, 2 * D), 0)
    col = jax.lax.broadcasted_iota(jnp.int32, (2 * D, 2 * D), 1)
    return jnp.where(2 * (row % D) + row // D == col, 1.0, 0.0)


def _mps_body(m2_ref, xft_ref, out_ref, alt_ref, gl_ref):
    j = pl.program_id(0)

    @pl.when(j == 0)
    def _init():
        row = jax.lax.broadcasted_iota(jnp.int32, (D, B), 0)
        alt_ref[...] = jnp.where(row == 0, 1.0, 0.0)
        rowg = jax.lax.broadcasted_iota(jnp.int32, (D, D), 0)
        colg = jax.lax.broadcasted_iota(jnp.int32, (D, D), 1)
        gl_ref[...] = jnp.where((rowg == 0) & (colg == 0), 1.0, 0.0)

    perm = _row_sort_perm()

    xblk = xft_ref[...]
    e0b = jnp.cos(0.5 * jnp.pi * xblk)
    e1b = jnp.sin(0.5 * jnp.pi * xblk)

    alt = alt_ref[...]
    gl = gl_ref[...]
    for s in range(S):
        slab = m2_ref[2 * D * s:2 * D * (s + 1), :]
        mv = jnp.dot(perm, slab,
                     preferred_element_type=jnp.float32)
        mcat = jnp.concatenate([mv[:D], mv[D:]], axis=1)

        vb = jnp.concatenate([alt * e0b[s:s + 1], alt * e1b[s:s + 1]], axis=0)
        alt = jax.lax.dot_general(
            mv, vb, (((0,), (0,)), ((), ())),
            preferred_element_type=jnp.float32)

        w = jnp.dot(gl, mcat,
                    preferred_element_type=jnp.float32)
        wv = jnp.concatenate([w[:, :D], w[:, D:]], axis=0)
        gl = jax.lax.dot_general(
            wv, mv, (((0,), (0,)), ((), ())),
            preferred_element_type=jnp.float32)
    alt_ref[...] = alt
    gl_ref[...] = gl

    @pl.when(j == NBLK - 1)
    def _():
        out_ref[0] = alt
        out_ref[1, :, :D] = gl


@functools.partial(jax.jit, static_argnames=("interpret",))
def kernel(x, MPS, interpret=False):
    xft = x.reshape(B, -1).T
    m2 = MPS.transpose(0, 1, 3, 2).reshape(N_SITES * 2 * D, D)

    buf = pl.pallas_call(
        _mps_body,
        grid=(NBLK,),
        in_specs=[
            pl.BlockSpec((S * 2 * D, D), lambda j: (j, 0)),
            pl.BlockSpec((S, B), lambda j: (j, 0)),
        ],
        out_specs=pl.BlockSpec((2, D, B), lambda j: (0, 0, 0)),
        out_shape=jax.ShapeDtypeStruct((2, D, B), jnp.float32),
        scratch_shapes=[
            pltpu.VMEM((D, B), jnp.float32),
            pltpu.VMEM((D, D), jnp.float32),
        ],
        compiler_params=pltpu.CompilerParams(
            dimension_semantics=("arbitrary",),
        ),
        interpret=interpret,
    )(m2, xft)

    amp = buf[0, 0, :]
    norm_sq = buf[1, 0, 0]
    return amp * amp / norm_sq
